# baseline (speedup 1.0000x reference)
import functools

import jax
import jax.numpy as jnp
from jax.experimental import pallas as pl

N = 4096
DIM = 6
NSP = 3
R = 0.5
HID = 128
MO = 64
DP = 8
BI = 512
NBLK = N // BI


def _dot_t(a, b):
    return jax.lax.dot_general(
        a, b, (((1,), (1,)), ((), ())), preferred_element_type=jnp.float32,
        precision=jax.lax.Precision.HIGHEST)


def _step_body(cur_blk, cur_full, bcol_blk, brow_full, w1p, b1r, w2, wor,
               out_blk):
    x_i = cur_blk[...]
    pre1 = jnp.dot(x_i.astype(jnp.bfloat16), w1p[...].astype(jnp.bfloat16),
                   preferred_element_type=jnp.float32) + b1r[...]

    col = jax.lax.broadcasted_iota(jnp.int32, (BI, DP), 1)
    pos_i = jnp.where(col < NSP, x_i, 0.0)
    sq_i = jnp.sum(pos_i * pos_i, axis=1, keepdims=True)
    bc_i = bcol_blk[...]
    bmin_i = jnp.min(bc_i)
    bmax_i = jnp.max(bc_i)
    ones_col = jnp.ones((BI, 1), jnp.float32)

    def jbody(j, c):
        x_j = cur_full[pl.ds(j * BI, BI), :]
        b_j = brow_full[:, pl.ds(j * BI, BI)]
        overlap = (jnp.min(b_j) <= bmax_i) & (jnp.max(b_j) >= bmin_i)

        def compute(_):
            pos_j = jnp.where(col < NSP, x_j, 0.0)
            sq_j = _dot_t(jnp.ones((1, DP), jnp.float32), pos_j * pos_j)
            dotmat = jax.lax.dot_general(
                pos_i.astype(jnp.bfloat16), pos_j.astype(jnp.bfloat16),
                (((1,), (1,)), ((), ())), preferred_element_type=jnp.float32)
            d2 = (sq_i + sq_j) - 2.0 * dotmat
            m = (d2 < R * R) & (bc_i == b_j)
            mf = jnp.where(m, 1.0, 0.0)
            return jnp.dot(mf, ones_col, preferred_element_type=jnp.float32,
                           precision=jax.lax.Precision.HIGHEST)

        return c + jax.lax.cond(overlap, compute,
                                lambda _: jnp.zeros((BI, 1), jnp.float32),
                                0)

    c = jax.lax.fori_loop(0, NBLK, jbody, jnp.zeros((BI, 1), jnp.float32))
    c = c - 1.0

    wo_f = wor[...].astype(jnp.bfloat16).astype(jnp.float32)
    dmsg = c * wo_f
    dh = jax.lax.dot_general(
        dmsg.astype(jnp.bfloat16), w2[...].astype(jnp.bfloat16),
        (((1,), (1,)), ((), ())), preferred_element_type=jnp.float32)
    dpre = jnp.where(pre1 > 0, dh, 0.0)
    dx = jax.lax.dot_general(
        dpre.astype(jnp.bfloat16), w1p[...].astype(jnp.bfloat16),
        (((1,), (1,)), ((), ())), preferred_element_type=jnp.float32)
    out_blk[...] = x_i - dx * 0.1


@functools.partial(jax.jit, static_argnames=())
def _one_step(cur_pad, bcol, brow, w1p, b1r, w2, wor):
    return pl.pallas_call(
        _step_body,
        grid=(NBLK,),
        in_specs=[
            pl.BlockSpec((BI, DP), lambda i: (i, 0)),
            pl.BlockSpec((N, DP), lambda i: (0, 0)),
            pl.BlockSpec((BI, 1), lambda i: (i, 0)),
            pl.BlockSpec((1, N), lambda i: (0, 0)),
            pl.BlockSpec((DP, HID), lambda i: (0, 0)),
            pl.BlockSpec((1, HID), lambda i: (0, 0)),
            pl.BlockSpec((HID, MO), lambda i: (0, 0)),
            pl.BlockSpec((1, MO), lambda i: (0, 0)),
        ],
        out_specs=pl.BlockSpec((BI, DP), lambda i: (i, 0)),
        out_shape=jax.ShapeDtypeStruct((N, DP), jnp.float32),
    )(cur_pad, cur_pad, bcol, brow, w1p, b1r, w2, wor)


def kernel(x, batch, steps, W1, b1, W2, b2, Wo, bo):
    cur_pad = jnp.pad(x, ((0, 0), (0, DP - DIM)))
    bcol = batch.reshape(N, 1)
    brow = batch.reshape(1, N)
    w1p = jnp.pad(W1, ((0, DP - DIM), (0, 0)))
    b1r = b1.reshape(1, HID)
    wor = Wo.reshape(1, MO)

    def step(_, cp):
        return _one_step(cp, bcol, brow, w1p, b1r, W2, wor)

    out = jax.lax.fori_loop(0, steps, step, cur_pad)
    return out[:, :DIM]
